# Initial kernel scaffold; baseline (speedup 1.0000x reference)
import jax
import jax.numpy as jnp
from jax.experimental import pallas as pl


def kernel(x, edge_index, batch, W1, b1, W2, b2, W3, b3, fc1_W, fc1_b, fc2_W, fc2_b):
    raise NotImplementedError("write your pallas kernel here")


# speedup vs baseline: 11.7279x; 11.7279x over previous
import functools

import jax
import jax.numpy as jnp
from jax import lax
from jax.experimental import pallas as pl
from jax.experimental.pallas import tpu as pltpu
from jax.experimental.pallas import tpu_sc as plsc

NN = 10000
EE = 320000
DIN = 128
F = 64
DOUT = 32
NG = 16

NC, NS = 2, 16
NW = NC * NS
ROWS_PT = 640
NPAD = NS * ROWS_PT
EPW = 10240
RPT = EPW // 128
EPAD = NW * EPW

_f32 = jnp.float32


@functools.cache
def _sc_kernels():
    mesh = plsc.VectorSubcoreMesh(
        core_axis_name="c", subcore_axis_name="s",
        num_cores=NC, num_subcores=NS,
    )

    @functools.partial(
        pl.kernel,
        out_type=jax.ShapeDtypeStruct((NC, NPAD, 16), _f32),
        mesh=mesh,
        compiler_params=pltpu.CompilerParams(use_tc_tiling_on_sc=False),
        scratch_types=[
            pltpu.VMEM((RPT, 128), jnp.int32),
            pltpu.VMEM((128, 16), _f32),
            pltpu.VMEM_SHARED((NPAD, 16), _f32),
        ],
    )
    def sc_deg(dst_hbm, zeros_hbm, ones_hbm, out_hbm, dst_v, rows_v, acc_sh):
        cid = lax.axis_index("c")
        sid = lax.axis_index("s")
        w = cid * NS + sid
        pltpu.sync_copy(dst_hbm.at[w], dst_v)
        pltpu.sync_copy(zeros_hbm, rows_v)
        for i in range(ROWS_PT // 128):
            pltpu.sync_copy(rows_v,
                            acc_sh.at[pl.ds(sid * ROWS_PT + i * 128, 128)])
        pltpu.sync_copy(ones_hbm, rows_v)
        plsc.subcore_barrier()

        @pl.loop(0, RPT)
        def _(j):
            pltpu.sync_copy(rows_v, acc_sh.at[dst_v.at[j]], add=True)

        plsc.subcore_barrier()
        pltpu.sync_copy(
            acc_sh.at[pl.ds(sid * ROWS_PT, ROWS_PT)],
            out_hbm.at[cid, pl.ds(sid * ROWS_PT, ROWS_PT)],
        )

    @functools.partial(
        pl.kernel,
        out_type=jax.ShapeDtypeStruct((NC, NPAD, F), _f32),
        mesh=mesh,
        compiler_params=pltpu.CompilerParams(use_tc_tiling_on_sc=False),
        scratch_types=[
            pltpu.VMEM((RPT, 128), jnp.int32),
            pltpu.VMEM((RPT, 128), jnp.int32),
            pltpu.VMEM((128, F), _f32),
            pltpu.VMEM_SHARED((NPAD, F), _f32),
            pltpu.SemaphoreType.DMA,
        ],
    )
    def sc_scatter(a_hbm, src_hbm, dst_hbm, zeros_hbm, out_hbm,
                   src_v, dst_v, rows_v, acc_sh, sem):
        cid = lax.axis_index("c")
        sid = lax.axis_index("s")
        w = cid * NS + sid
        pltpu.sync_copy(src_hbm.at[w], src_v)
        pltpu.sync_copy(dst_hbm.at[w], dst_v)
        pltpu.sync_copy(zeros_hbm, rows_v)
        for i in range(ROWS_PT // 128):
            pltpu.sync_copy(rows_v,
                            acc_sh.at[pl.ds(sid * ROWS_PT + i * 128, 128)])
        plsc.subcore_barrier()

        @pl.loop(0, RPT)
        def _(j):
            pltpu.async_copy(a_hbm.at[src_v.at[j]], rows_v, sem).wait()
            pltpu.sync_copy(rows_v, acc_sh.at[dst_v.at[j]], add=True)

        plsc.subcore_barrier()
        pltpu.sync_copy(
            acc_sh.at[pl.ds(sid * ROWS_PT, ROWS_PT)],
            out_hbm.at[cid, pl.ds(sid * ROWS_PT, ROWS_PT)],
        )

    return sc_deg, sc_scatter


def _prep_body(x_ref, w_ref, deg_ref, a_ref, dinv_ref):
    degs = deg_ref[...]
    deg = degs[0, :NN, 0:1] + degs[1, :NN, 0:1] + 1.0
    dinv = lax.rsqrt(deg)
    u = jnp.dot(x_ref[...], w_ref[...], preferred_element_type=_f32)
    a_ref[...] = u * dinv
    dinv_ref[...] = dinv


_tc_prep = pl.pallas_call(
    _prep_body,
    out_shape=(
        jax.ShapeDtypeStruct((NN, F), _f32),
        jax.ShapeDtypeStruct((NN, 1), _f32),
    ),
)


def _mid_body(s_ref, a_ref, dinv_ref, b_ref, w_ref, out_ref):
    s = s_ref[0, :NN, :] + s_ref[1, :NN, :]
    dinv = dinv_ref[...]
    h = jax.nn.relu(dinv * (s + a_ref[...]) + b_ref[...])
    out_ref[...] = jnp.dot(h, w_ref[...], preferred_element_type=_f32) * dinv


_tc_mid = pl.pallas_call(
    _mid_body,
    out_shape=jax.ShapeDtypeStruct((NN, F), _f32),
)


def _final_body(s_ref, a_ref, dinv_ref, b_ref, batch_ref, fc1w_ref, fc1b_ref,
                fc2w_ref, fc2b_ref, out_ref):
    s = s_ref[0, :NN, :] + s_ref[1, :NN, :]
    h = dinv_ref[...] * (s + a_ref[...]) + b_ref[...]
    gids = lax.broadcasted_iota(jnp.int32, (NN, NG), 1)
    onehot = (batch_ref[...] == gids).astype(_f32)
    g = lax.dot_general(onehot, h, (((0,), (0,)), ((), ())),
                        preferred_element_type=_f32)
    t = jax.nn.relu(jnp.dot(g, fc1w_ref[...], preferred_element_type=_f32)
                    + fc1b_ref[...])
    scores = (jnp.dot(t, fc2w_ref[...], preferred_element_type=_f32)
              + fc2b_ref[...])
    m = jnp.max(scores, axis=-1, keepdims=True)
    lse = jnp.log(jnp.sum(jnp.exp(scores - m), axis=-1, keepdims=True)) + m
    out_ref[...] = scores - lse


_tc_final = pl.pallas_call(
    _final_body,
    out_shape=jax.ShapeDtypeStruct((NG, DOUT), _f32),
)


def kernel(x, edge_index, batch, W1, b1, W2, b2, W3, b3,
           fc1_W, fc1_b, fc2_W, fc2_b):
    src = edge_index[0]
    dst = edge_index[1]
    srcp = jnp.concatenate(
        [src, jnp.zeros((EPAD - EE,), jnp.int32)]).reshape(NW, RPT, 128)
    dstp = jnp.concatenate(
        [dst, jnp.full((EPAD - EE,), NN, jnp.int32)]).reshape(NW, RPT, 128)
    zeros16 = jnp.zeros((128, 16), _f32)
    ones16 = jnp.ones((128, 16), _f32)
    zeros64 = jnp.zeros((128, F), _f32)

    sc_deg, sc_scatter = _sc_kernels()
    degs = sc_deg(dstp, zeros16, ones16)
    a1, dinv = _tc_prep(x, W1, degs)
    s1 = sc_scatter(a1, srcp, dstp, zeros64)
    a2 = _tc_mid(s1, a1, dinv, b1.reshape(1, F), W2)
    s2 = sc_scatter(a2, srcp, dstp, zeros64)
    a3 = _tc_mid(s2, a2, dinv, b2.reshape(1, F), W3)
    s3 = sc_scatter(a3, srcp, dstp, zeros64)
    out = _tc_final(s3, a3, dinv, b3.reshape(1, F), batch.reshape(NN, 1),
                    fc1_W, fc1_b.reshape(1, F), fc2_W, fc2_b.reshape(1, DOUT))
    return out
